# Initial kernel scaffold; baseline (speedup 1.0000x reference)
import jax
import jax.numpy as jnp
from jax.experimental import pallas as pl


def kernel(input, table):
    raise NotImplementedError("write your pallas kernel here")


# speedup vs baseline: 1.5009x; 1.5009x over previous
import functools

import jax
import jax.numpy as jnp
from jax import lax
from jax.experimental import pallas as pl
from jax.experimental.pallas import tpu as pltpu
from jax.experimental.pallas import tpu_sc as plsc

EMBD_DIM = 32
BATCH = 4096
HIST = 200
B_TOTAL = BATCH * HIST

NUM_CORES = 2
NUM_SUBCORES = 16
NW = NUM_CORES * NUM_SUBCORES
B_PER_W = B_TOTAL // NW
CHUNK = 2560
NCHUNK = B_PER_W // CHUNK


def _build():
    mesh = plsc.VectorSubcoreMesh(core_axis_name="c", subcore_axis_name="s")

    @functools.partial(
        pl.kernel,
        mesh=mesh,
        out_type=jax.ShapeDtypeStruct((B_TOTAL, EMBD_DIM), jnp.float32),
        scratch_types=[
            pltpu.VMEM((CHUNK,), jnp.int32),
            pltpu.VMEM((CHUNK, EMBD_DIM), jnp.float32),
            pltpu.SemaphoreType.DMA,
        ],
        compiler_params=pltpu.CompilerParams(use_tc_tiling_on_sc=False),
    )
    def gather_kernel(idx_hbm, table_hbm, out_hbm, idx_v, rows_v, sem):
        wid = lax.axis_index("s") * NUM_CORES + lax.axis_index("c")
        base0 = wid * B_PER_W

        def body(i, carry):
            base = base0 + i * CHUNK
            pltpu.sync_copy(idx_hbm.at[pl.ds(base, CHUNK)], idx_v)
            pltpu.async_copy(table_hbm.at[idx_v], rows_v, sem).wait()
            pltpu.sync_copy(rows_v, out_hbm.at[pl.ds(base, CHUNK)])
            return carry

        lax.fori_loop(0, NCHUNK, body, 0)

    return gather_kernel


_gather = _build()


def kernel(input, table):
    idx = input.reshape(B_TOTAL).astype(jnp.int32)
    out = _gather(idx, table)
    return out.reshape(BATCH, HIST, EMBD_DIM)


# speedup vs baseline: 1.5117x; 1.0072x over previous
import functools

import jax
import jax.numpy as jnp
from jax import lax
from jax.experimental import pallas as pl
from jax.experimental.pallas import tpu as pltpu
from jax.experimental.pallas import tpu_sc as plsc

EMBD_DIM = 32
BATCH = 4096
HIST = 200
B_TOTAL = BATCH * HIST

NUM_CORES = 2
NUM_SUBCORES = 16
NW = NUM_CORES * NUM_SUBCORES
B_PER_W = B_TOTAL // NW
CHUNK = 1280
NCHUNK = B_PER_W // CHUNK
NBUF = 2


def _build():
    mesh = plsc.VectorSubcoreMesh(core_axis_name="c", subcore_axis_name="s")

    @functools.partial(
        pl.kernel,
        mesh=mesh,
        out_type=jax.ShapeDtypeStruct((B_TOTAL, EMBD_DIM), jnp.float32),
        scratch_types=[
            pltpu.VMEM((NCHUNK, CHUNK), jnp.int32),
            [pltpu.VMEM((CHUNK, EMBD_DIM), jnp.float32) for _ in range(NBUF)],
            [pltpu.SemaphoreType.DMA for _ in range(NBUF)],
            [pltpu.SemaphoreType.DMA for _ in range(NBUF)],
        ],
        compiler_params=pltpu.CompilerParams(use_tc_tiling_on_sc=False),
    )
    def gather_kernel(idx_hbm, table_hbm, out_hbm, idx_v, bufs, gsems, ssems):
        wid = lax.axis_index("s") * NUM_CORES + lax.axis_index("c")
        base0 = wid * B_PER_W
        pltpu.sync_copy(idx_hbm.at[wid], idx_v)

        def start_gather(i):
            b = i % NBUF
            return pltpu.async_copy(table_hbm.at[idx_v.at[i]], bufs[b], gsems[b])

        def start_store(i):
            b = i % NBUF
            return pltpu.async_copy(
                bufs[b], out_hbm.at[pl.ds(base0 + i * CHUNK, CHUNK)], ssems[b]
            )

        gathers = [None] * NCHUNK
        stores = [None] * NCHUNK
        gathers[0] = start_gather(0)
        for i in range(NCHUNK):
            if i + 1 < NCHUNK:
                if i + 1 >= NBUF:
                    stores[i + 1 - NBUF].wait()
                gathers[i + 1] = start_gather(i + 1)
            gathers[i].wait()
            stores[i] = start_store(i)
        for i in range(NCHUNK - NBUF, NCHUNK):
            stores[i].wait()

    return gather_kernel


_gather = _build()


def kernel(input, table):
    idx = input.reshape(NW, NCHUNK, CHUNK).astype(jnp.int32)
    out = _gather(idx, table)
    return out.reshape(BATCH, HIST, EMBD_DIM)
